# baseline (speedup 1.0000x reference)
import jax
import jax.numpy as jnp
from jax.experimental import pallas as pl
from jax.experimental.pallas import tpu as pltpu

_BK = 128


def _gcnn_kernel(adj_hbm, x_ref, w_ref, b_ref, out_ref,
                 buf0, buf1, ab_ref, sem0, sem1):
    n = adj_hbm.shape[0]
    num_blocks = n // _BK
    bufs = (buf0, buf1)
    sems = (sem0, sem1)

    def block_copy(k):
        return pltpu.make_async_copy(
            adj_hbm.at[pl.ds(k * _BK, _BK), :], bufs[k % 2], sems[k % 2])

    block_copy(0).start()
    cs = jnp.zeros((1, n), jnp.int32)
    for k in range(num_blocks):
        if k + 1 < num_blocks:
            block_copy(k + 1).start()
        block_copy(k).wait()
        blk = bufs[k % 2][...]
        cs = cs + jnp.sum(blk, axis=0, keepdims=True)
        ab_ref[k * _BK:(k + 1) * _BK, :] = blk.astype(jnp.bfloat16)

    d = jax.lax.rsqrt(cs.astype(jnp.float32) + 1.0)
    dc = d.reshape(-1, 1)
    xw = jnp.dot(x_ref[...], w_ref[...], preferred_element_type=jnp.float32)
    y = xw * dc
    z = jax.lax.dot_general(ab_ref[...], y.astype(jnp.bfloat16),
                            (((0,), (0,)), ((), ())),
                            preferred_element_type=jnp.float32)
    out_ref[...] = (z + y) * dc + b_ref[...]


def kernel(batch_inputs, batch_graph, W, b):
    n, f = batch_inputs.shape
    fo = W.shape[1]
    return pl.pallas_call(
        _gcnn_kernel,
        in_specs=[
            pl.BlockSpec(memory_space=pl.ANY),
            pl.BlockSpec((n, f), lambda: (0, 0)),
            pl.BlockSpec((f, fo), lambda: (0, 0)),
            pl.BlockSpec((1, fo), lambda: (0, 0)),
        ],
        out_specs=pl.BlockSpec((n, fo), lambda: (0, 0)),
        scratch_shapes=[
            pltpu.VMEM((_BK, n), jnp.int32),
            pltpu.VMEM((_BK, n), jnp.int32),
            pltpu.VMEM((n, n), jnp.bfloat16),
            pltpu.SemaphoreType.DMA,
            pltpu.SemaphoreType.DMA,
        ],
        out_shape=jax.ShapeDtypeStruct((n, fo), batch_inputs.dtype),
    )(batch_graph, batch_inputs, W, b.reshape(1, -1))


# speedup vs baseline: 1.4841x; 1.4841x over previous
import jax
import jax.numpy as jnp
from jax.experimental import pallas as pl
from jax.experimental.pallas import tpu as pltpu

_BK = 512


def _gcnn_kernel(adj_hbm, x_ref, w_ref, b_ref, out_ref,
                 buf0, buf1, ab_ref, sem0, sem1):
    n = adj_hbm.shape[0]
    num_blocks = n // _BK
    bufs = (buf0, buf1)
    sems = (sem0, sem1)

    def block_copy(k):
        return pltpu.make_async_copy(
            adj_hbm.at[pl.ds(k * _BK, _BK), :], bufs[k % 2], sems[k % 2])

    block_copy(0).start()
    cs = jnp.zeros((1, n), jnp.int32)
    for k in range(num_blocks):
        if k + 1 < num_blocks:
            block_copy(k + 1).start()
        block_copy(k).wait()
        blk = bufs[k % 2][...]
        cs = cs + jnp.sum(blk, axis=0, keepdims=True)
        ab_ref[k * _BK:(k + 1) * _BK, :] = blk.astype(jnp.bfloat16)

    d = jax.lax.rsqrt(cs.astype(jnp.float32) + 1.0)
    dc = d.reshape(-1, 1)
    xw = jnp.dot(x_ref[...], w_ref[...], preferred_element_type=jnp.float32)
    y = xw * dc
    z = jax.lax.dot_general(ab_ref[...], y.astype(jnp.bfloat16),
                            (((0,), (0,)), ((), ())),
                            preferred_element_type=jnp.float32)
    out_ref[...] = (z + y) * dc + b_ref[...]


def kernel(batch_inputs, batch_graph, W, b):
    n, f = batch_inputs.shape
    fo = W.shape[1]
    return pl.pallas_call(
        _gcnn_kernel,
        in_specs=[
            pl.BlockSpec(memory_space=pl.ANY),
            pl.BlockSpec((n, f), lambda: (0, 0)),
            pl.BlockSpec((f, fo), lambda: (0, 0)),
            pl.BlockSpec((1, fo), lambda: (0, 0)),
        ],
        out_specs=pl.BlockSpec((n, fo), lambda: (0, 0)),
        scratch_shapes=[
            pltpu.VMEM((_BK, n), jnp.int32),
            pltpu.VMEM((_BK, n), jnp.int32),
            pltpu.VMEM((n, n), jnp.bfloat16),
            pltpu.SemaphoreType.DMA,
            pltpu.SemaphoreType.DMA,
        ],
        out_shape=jax.ShapeDtypeStruct((n, fo), batch_inputs.dtype),
    )(batch_graph, batch_inputs, W, b.reshape(1, -1))


# speedup vs baseline: 1.7942x; 1.2090x over previous
import jax
import jax.numpy as jnp
from jax.experimental import pallas as pl


def _gcnn_kernel(adj_ref, x_ref, w_ref, b_ref, out_ref):
    ai = adj_ref[...]
    deg = jnp.sum(ai, axis=0, keepdims=True)
    d = jax.lax.rsqrt(deg.astype(jnp.float32) + 1.0)
    dc = d.reshape(-1, 1)
    xw = jnp.dot(x_ref[...], w_ref[...], preferred_element_type=jnp.float32)
    y = xw * dc
    zt = jnp.dot(y.astype(jnp.bfloat16).T, ai.astype(jnp.bfloat16),
                 preferred_element_type=jnp.float32)
    out_ref[...] = (zt.T + y) * dc + b_ref[...]


def kernel(batch_inputs, batch_graph, W, b):
    n, f = batch_inputs.shape
    return pl.pallas_call(
        _gcnn_kernel,
        out_shape=jax.ShapeDtypeStruct((n, W.shape[1]), batch_inputs.dtype),
    )(batch_graph, batch_inputs, W, b.reshape(1, -1))
